# baseline (speedup 1.0000x reference)
import jax
import jax.numpy as jnp
from jax.experimental import pallas as pl

STEPS = 2
NSPLIT = 4


def _encoder_kernel(*refs):
    adj_refs = refs[:NSPLIT]
    (x_ref, lin0_w_ref, lin0_b_ref, gin_w_ref, gin_b_ref, w_ih_ref,
     w_hh_ref, b_ih_ref, b_hh_ref, out_ref) = refs[NSPLIT:]
    f32 = jnp.float32
    bf16 = jnp.bfloat16
    adj_chunks = [r[0].astype(bf16) for r in adj_refs]
    x = x_ref[0]

    out_t = jax.nn.relu(
        jax.lax.dot_general(lin0_w_ref[...], x,
                            (((1,), (1,)), ((), ())),
                            preferred_element_type=f32)
        + lin0_b_ref[...])
    h_t = out_t
    H = out_t.shape[0]

    for _ in range(STEPS):
        out_b = out_t.astype(bf16)
        agg_t = jnp.concatenate(
            [jax.lax.dot_general(out_b, c, (((1,), (0,)), ((), ())),
                                 preferred_element_type=f32)
             for c in adj_chunks], axis=1)
        m_t = jax.nn.relu(
            jax.lax.dot_general(gin_w_ref[...], out_t + agg_t,
                                (((1,), (0,)), ((), ())),
                                preferred_element_type=f32)
            + gin_b_ref[...])
        gi = jax.lax.dot_general(w_ih_ref[...], m_t,
                                 (((1,), (0,)), ((), ())),
                                 preferred_element_type=f32) + b_ih_ref[...]
        gh = jax.lax.dot_general(w_hh_ref[...], h_t,
                                 (((1,), (0,)), ((), ())),
                                 preferred_element_type=f32) + b_hh_ref[...]
        r = jax.nn.sigmoid(gi[:H] + gh[:H])
        z = jax.nn.sigmoid(gi[H:2 * H] + gh[H:2 * H])
        n = jnp.tanh(gi[2 * H:] + r * gh[2 * H:])
        out_t = (1.0 - z) * n + z * h_t
        h_t = out_t

    out_ref[0] = out_t


def kernel(adj, n_feat, lin0_w, lin0_b, gin_w, gin_b, gru_w_ih, gru_w_hh,
           gru_b_ih, gru_b_hh):
    B, N, FT = n_feat.shape
    H = lin0_w.shape[0]

    full = lambda shape: pl.BlockSpec(shape, lambda b: (0,) * len(shape))
    adj_specs = [
        pl.BlockSpec((1, N, N // NSPLIT), lambda b, c=c: (b, 0, c))
        for c in range(NSPLIT)
    ]
    out3 = pl.pallas_call(
        _encoder_kernel,
        grid=(B,),
        in_specs=adj_specs + [
            pl.BlockSpec((1, N, FT), lambda b: (b, 0, 0)),
            full((H, FT)),
            full((H, 1)),
            full((H, H)),
            full((H, 1)),
            full((3 * H, H)),
            full((3 * H, H)),
            full((3 * H, 1)),
            full((3 * H, 1)),
        ],
        out_specs=pl.BlockSpec((1, H, N), lambda b: (b, 0, 0)),
        out_shape=jax.ShapeDtypeStruct((B, H, N), jnp.float32),
    )(*([adj] * NSPLIT), n_feat, lin0_w, lin0_b.reshape(H, 1), gin_w,
      gin_b.reshape(H, 1), gru_w_ih, gru_w_hh, gru_b_ih.reshape(3 * H, 1),
      gru_b_hh.reshape(3 * H, 1))
    return out3.transpose(0, 2, 1).reshape(B * N, H)


# speedup vs baseline: 1.0084x; 1.0084x over previous
import jax
import jax.numpy as jnp
from jax.experimental import pallas as pl

STEPS = 2
NSPLIT = 4


def _encoder_kernel(*refs):
    adj_refs = refs[:NSPLIT]
    (x_ref, lin0_w_ref, lin0_b_ref, gin_w_ref, gin_b_ref, w_ih_ref,
     w_hh_ref, b_ih_ref, b_hh_ref, out_ref) = refs[NSPLIT:]
    f32 = jnp.float32
    bf16 = jnp.bfloat16
    adj_chunks = [r[0].astype(bf16) for r in adj_refs]
    x = x_ref[0]

    out_t = jax.nn.relu(
        jax.lax.dot_general(lin0_w_ref[...], x,
                            (((1,), (1,)), ((), ())),
                            preferred_element_type=f32)
        + lin0_b_ref[...])
    h_t = out_t
    H = out_t.shape[0]

    for _ in range(STEPS):
        out_b = out_t.astype(bf16)
        nu = adj_chunks[0].shape[0]
        agg_t = sum(
            jax.lax.dot_general(out_b[:, i * nu:(i + 1) * nu], c,
                                (((1,), (0,)), ((), ())),
                                preferred_element_type=f32)
            for i, c in enumerate(adj_chunks))
        m_t = jax.nn.relu(
            jax.lax.dot_general(gin_w_ref[...], out_t + agg_t,
                                (((1,), (0,)), ((), ())),
                                preferred_element_type=f32)
            + gin_b_ref[...])
        gi = jax.lax.dot_general(w_ih_ref[...], m_t,
                                 (((1,), (0,)), ((), ())),
                                 preferred_element_type=f32) + b_ih_ref[...]
        gh = jax.lax.dot_general(w_hh_ref[...], h_t,
                                 (((1,), (0,)), ((), ())),
                                 preferred_element_type=f32) + b_hh_ref[...]
        r = jax.nn.sigmoid(gi[:H] + gh[:H])
        z = jax.nn.sigmoid(gi[H:2 * H] + gh[H:2 * H])
        n = jnp.tanh(gi[2 * H:] + r * gh[2 * H:])
        out_t = (1.0 - z) * n + z * h_t
        h_t = out_t

    out_ref[0] = out_t


def kernel(adj, n_feat, lin0_w, lin0_b, gin_w, gin_b, gru_w_ih, gru_w_hh,
           gru_b_ih, gru_b_hh):
    B, N, FT = n_feat.shape
    H = lin0_w.shape[0]

    full = lambda shape: pl.BlockSpec(shape, lambda b: (0,) * len(shape))
    adj_specs = [
        pl.BlockSpec((1, N // NSPLIT, N), lambda b, c=c: (b, c, 0))
        for c in range(NSPLIT)
    ]
    out3 = pl.pallas_call(
        _encoder_kernel,
        grid=(B,),
        in_specs=adj_specs + [
            pl.BlockSpec((1, N, FT), lambda b: (b, 0, 0)),
            full((H, FT)),
            full((H, 1)),
            full((H, H)),
            full((H, 1)),
            full((3 * H, H)),
            full((3 * H, H)),
            full((3 * H, 1)),
            full((3 * H, 1)),
        ],
        out_specs=pl.BlockSpec((1, H, N), lambda b: (b, 0, 0)),
        out_shape=jax.ShapeDtypeStruct((B, H, N), jnp.float32),
    )(*([adj] * NSPLIT), n_feat, lin0_w, lin0_b.reshape(H, 1), gin_w,
      gin_b.reshape(H, 1), gru_w_ih, gru_w_hh, gru_b_ih.reshape(3 * H, 1),
      gru_b_hh.reshape(3 * H, 1))
    return out3.transpose(0, 2, 1).reshape(B * N, H)


# speedup vs baseline: 1.0656x; 1.0567x over previous
import jax
import jax.numpy as jnp
from jax.experimental import pallas as pl
from jax.experimental.pallas import tpu as pltpu

STEPS = 2
NSPLIT = 1


def _encoder_kernel(*refs):
    adj_refs = refs[:NSPLIT]
    (x_ref, lin0_w_ref, lin0_b_ref, gin_w_ref, gin_b_ref, w_ih_ref,
     w_hh_ref, b_ih_ref, b_hh_ref, out_ref) = refs[NSPLIT:]
    f32 = jnp.float32
    bf16 = jnp.bfloat16
    adj_chunks = [r[0].astype(bf16) for r in adj_refs]
    x = x_ref[0]

    out_t = jax.nn.relu(
        jax.lax.dot_general(lin0_w_ref[...], x,
                            (((1,), (1,)), ((), ())),
                            preferred_element_type=f32)
        + lin0_b_ref[...])
    h_t = out_t
    H = out_t.shape[0]

    for _ in range(STEPS):
        out_b = out_t.astype(bf16)
        nu = adj_chunks[0].shape[0]
        agg_t = sum(
            jax.lax.dot_general(out_b[:, i * nu:(i + 1) * nu], c,
                                (((1,), (0,)), ((), ())),
                                preferred_element_type=f32)
            for i, c in enumerate(adj_chunks))
        m_t = jax.nn.relu(
            jax.lax.dot_general(gin_w_ref[...], out_t + agg_t,
                                (((1,), (0,)), ((), ())),
                                preferred_element_type=f32)
            + gin_b_ref[...])
        gi = jax.lax.dot_general(w_ih_ref[...], m_t,
                                 (((1,), (0,)), ((), ())),
                                 preferred_element_type=f32) + b_ih_ref[...]
        gh = jax.lax.dot_general(w_hh_ref[...], h_t,
                                 (((1,), (0,)), ((), ())),
                                 preferred_element_type=f32) + b_hh_ref[...]
        r = jax.nn.sigmoid(gi[:H] + gh[:H])
        z = jax.nn.sigmoid(gi[H:2 * H] + gh[H:2 * H])
        n = jnp.tanh(gi[2 * H:] + r * gh[2 * H:])
        out_t = (1.0 - z) * n + z * h_t
        h_t = out_t

    out_ref[0] = out_t


def kernel(adj, n_feat, lin0_w, lin0_b, gin_w, gin_b, gru_w_ih, gru_w_hh,
           gru_b_ih, gru_b_hh):
    B, N, FT = n_feat.shape
    H = lin0_w.shape[0]

    full = lambda shape: pl.BlockSpec(shape, lambda b: (0,) * len(shape))
    adj_specs = [
        pl.BlockSpec((1, N // NSPLIT, N), lambda b, c=c: (b, c, 0))
        for c in range(NSPLIT)
    ]
    out3 = pl.pallas_call(
        _encoder_kernel,
        grid=(B,),
        in_specs=adj_specs + [
            pl.BlockSpec((1, N, FT), lambda b: (b, 0, 0)),
            full((H, FT)),
            full((H, 1)),
            full((H, H)),
            full((H, 1)),
            full((3 * H, H)),
            full((3 * H, H)),
            full((3 * H, 1)),
            full((3 * H, 1)),
        ],
        out_specs=pl.BlockSpec((1, H, N), lambda b: (b, 0, 0)),
        out_shape=jax.ShapeDtypeStruct((B, H, N), jnp.float32),
        compiler_params=pltpu.CompilerParams(
            dimension_semantics=("parallel",)),
    )(*([adj] * NSPLIT), n_feat, lin0_w, lin0_b.reshape(H, 1), gin_w,
      gin_b.reshape(H, 1), gru_w_ih, gru_w_hh, gru_b_ih.reshape(3 * H, 1),
      gru_b_hh.reshape(3 * H, 1))
    return out3.transpose(0, 2, 1).reshape(B * N, H)
